# Initial kernel scaffold; baseline (speedup 1.0000x reference)
import jax
import jax.numpy as jnp
from jax.experimental import pallas as pl


def kernel(inputs):
    raise NotImplementedError("write your pallas kernel here")


# speedup vs baseline: 4.6393x; 4.6393x over previous
import functools

import jax
import jax.numpy as jnp
import numpy as np
from jax import lax
from jax.experimental import pallas as pl
from jax.experimental.pallas import tpu as pltpu
from jax.experimental.pallas import tpu_sc as plsc

_STRIDES = 2
_H = 112
_W = 112
_C = 96
_B = 4
_HD = _H * _STRIDES
_WD = _W * _STRIDES
_ROW_IN = _W * _C
_ROW_OUT = _WD * _C
_NW = 32
_ROWS_PER_W = (_B * _HD) // _NW
_WPB = _HD // _ROWS_PER_W
_CG = _C // 16
_TBL = 240


def _interp_tables(src_size):
    s = float(src_size - 1)
    d = np.arange(2 * src_size, dtype=np.float64)
    low = (d - 1.0) / 1.5
    high = (d - 1.0 + 0.5 - (s - 1.0) * 2.0) / 1.5 + (s - 1.0)
    mid = (d - 1.0 + 0.5) / 2.0
    src = np.where(d < 2.5, low, np.where(d > 1.0 + (s - 1.0) * 2.0 - 0.5, high, mid))
    r0 = np.floor(src).astype(np.int64)
    fr = src - r0
    w0 = (1.0 - fr) * ((r0 >= 0) & (r0 < src_size))
    w1 = fr * ((r0 + 1 >= 0) & (r0 + 1 < src_size))
    base = np.clip(r0, 0, src_size - 2)
    tap0 = np.select([r0 < 0, r0 > src_size - 2], [w1, 0.0], w0)
    tap1 = np.select([r0 < 0, r0 > src_size - 2], [0.0, w0], w1)
    return (base.astype(np.int32), tap0.astype(np.float32),
            tap1.astype(np.float32))


_R0_NP, _W0_NP, _W1_NP = _interp_tables(_H)
_C0_NP, _V0_NP, _V1_NP = _interp_tables(_W)


def _pad_tbl(a):
    return np.pad(a, (0, _TBL - a.shape[0]))


_EDGE_COLS = [(j, int(_C0_NP[j]), float(_V0_NP[j]), float(_V1_NP[j]))
              for j in (0, 1, 2, _WD - 3, _WD - 2, _WD - 1)]


@functools.partial(
    pl.kernel,
    mesh=plsc.VectorSubcoreMesh(core_axis_name="c", subcore_axis_name="s"),
    out_type=jax.ShapeDtypeStruct((_B * _HD, _ROW_OUT), jnp.float32),
    scratch_types=[
        pltpu.VMEM((_ROW_IN,), jnp.float32),
        pltpu.VMEM((_ROW_IN,), jnp.float32),
        pltpu.VMEM((_ROW_IN,), jnp.float32),
        pltpu.VMEM((_ROW_OUT,), jnp.float32),
        pltpu.VMEM((_TBL,), jnp.int32),
        pltpu.VMEM((_TBL,), jnp.float32),
        pltpu.VMEM((_TBL,), jnp.float32),
    ],
)
def _upsample_sc(x_hbm, r0_hbm, w0_hbm, w1_hbm,
                 out_hbm, row_a, row_b, row_t, row_o, r0_v, w0_v, w1_v):
    cid = lax.axis_index("c")
    sid = lax.axis_index("s")
    wid = sid * 2 + cid
    batch = wid // _WPB
    i_base = (wid % _WPB) * _ROWS_PER_W
    in_base = batch * _H
    out_base = batch * _HD

    pltpu.sync_copy(r0_hbm, r0_v)
    pltpu.sync_copy(w0_hbm, w0_v)
    pltpu.sync_copy(w1_hbm, w1_v)

    def per_row(n, carry):
        i = i_base + n
        r0 = r0_v[pl.ds(i, 16)][0]
        w0 = w0_v[pl.ds(i, 16)][0]
        w1 = w1_v[pl.ds(i, 16)][0]
        pltpu.sync_copy(x_hbm.at[in_base + r0], row_a)
        pltpu.sync_copy(x_hbm.at[in_base + r0 + 1], row_b)

        def blend(k, c2):
            o = k * _C
            for g in range(_CG):
                sl = pl.ds(o + g * 16, 16)
                row_t[sl] = w0 * row_a[sl] + w1 * row_b[sl]
            return c2

        lax.fori_loop(0, _W, blend, 0, unroll=4)

        def colpair(m, c2):
            k = m + 2
            t0 = (k - 1) * _C
            t1 = k * _C
            o_odd = (2 * k - 1) * _C
            o_even = 2 * k * _C
            for g in range(_CG):
                gg = g * 16
                a = row_t[pl.ds(t0 + gg, 16)]
                b = row_t[pl.ds(t1 + gg, 16)]
                row_o[pl.ds(o_odd + gg, 16)] = 0.75 * a + 0.25 * b
                row_o[pl.ds(o_even + gg, 16)] = 0.25 * a + 0.75 * b
            return c2

        lax.fori_loop(0, _W - 3, colpair, 0, unroll=4)

        for j, cb, v0, v1 in _EDGE_COLS:
            for g in range(_CG):
                gg = g * 16
                row_o[pl.ds(j * _C + gg, 16)] = (
                    v0 * row_t[pl.ds(cb * _C + gg, 16)]
                    + v1 * row_t[pl.ds((cb + 1) * _C + gg, 16)])

        pltpu.sync_copy(row_o, out_hbm.at[out_base + i])
        return carry

    lax.fori_loop(0, _ROWS_PER_W, per_row, 0)


def kernel(inputs):
    x = inputs.reshape(_B * _H, _ROW_IN)
    out = _upsample_sc(
        x,
        jnp.asarray(_pad_tbl(_R0_NP)),
        jnp.asarray(_pad_tbl(_W0_NP)),
        jnp.asarray(_pad_tbl(_W1_NP)),
    )
    return out.reshape(_B, _HD, _WD, _C)


# speedup vs baseline: 8.8742x; 1.9128x over previous
import functools

import jax
import jax.numpy as jnp
import numpy as np
from jax import lax
from jax.experimental import pallas as pl
from jax.experimental.pallas import tpu as pltpu
from jax.experimental.pallas import tpu_sc as plsc

_STRIDES = 2
_H = 112
_W = 112
_C = 96
_B = 4
_HD = _H * _STRIDES
_WD = _W * _STRIDES
_ROW_IN = _W * _C
_ROW_OUT = _WD * _C
_NW = 32
_ROWS_PER_W = (_B * _HD) // _NW
_WPB = _HD // _ROWS_PER_W
_CG = _C // 16
_TBL = 240


def _interp_tables(src_size):
    s = float(src_size - 1)
    d = np.arange(2 * src_size, dtype=np.float64)
    low = (d - 1.0) / 1.5
    high = (d - 1.0 + 0.5 - (s - 1.0) * 2.0) / 1.5 + (s - 1.0)
    mid = (d - 1.0 + 0.5) / 2.0
    src = np.where(d < 2.5, low, np.where(d > 1.0 + (s - 1.0) * 2.0 - 0.5, high, mid))
    r0 = np.floor(src).astype(np.int64)
    fr = src - r0
    w0 = (1.0 - fr) * ((r0 >= 0) & (r0 < src_size))
    w1 = fr * ((r0 + 1 >= 0) & (r0 + 1 < src_size))
    base = np.clip(r0, 0, src_size - 2)
    tap0 = np.select([r0 < 0, r0 > src_size - 2], [w1, 0.0], w0)
    tap1 = np.select([r0 < 0, r0 > src_size - 2], [0.0, w0], w1)
    return (base.astype(np.int32), tap0.astype(np.float32),
            tap1.astype(np.float32))


_R0_NP, _W0_NP, _W1_NP = _interp_tables(_H)
_C0_NP, _V0_NP, _V1_NP = _interp_tables(_W)


def _pad_tbl(a):
    return np.pad(a, (0, _TBL - a.shape[0]))


_EDGE_COLS = [(j, int(_C0_NP[j]), float(_V0_NP[j]), float(_V1_NP[j]))
              for j in (0, 1, 2, _WD - 3, _WD - 2, _WD - 1)]


_PAIRS = _ROWS_PER_W // 2


@functools.partial(
    pl.kernel,
    mesh=plsc.VectorSubcoreMesh(core_axis_name="c", subcore_axis_name="s"),
    out_type=jax.ShapeDtypeStruct((_B * _HD, _ROW_OUT), jnp.float32),
    scratch_types=[
        pltpu.VMEM((_ROW_IN,), jnp.float32),
        pltpu.VMEM((_ROW_IN,), jnp.float32),
        pltpu.VMEM((_ROW_IN,), jnp.float32),
        pltpu.VMEM((_ROW_IN,), jnp.float32),
        pltpu.VMEM((_ROW_IN,), jnp.float32),
        pltpu.VMEM((_ROW_OUT,), jnp.float32),
        pltpu.VMEM((_ROW_OUT,), jnp.float32),
        pltpu.VMEM((_TBL,), jnp.int32),
        pltpu.VMEM((_TBL,), jnp.float32),
        pltpu.VMEM((_TBL,), jnp.float32),
        pltpu.SemaphoreType.DMA,
        pltpu.SemaphoreType.DMA,
        pltpu.SemaphoreType.DMA,
        pltpu.SemaphoreType.DMA,
    ],
)
def _upsample_sc(x_hbm, r0_hbm, w0_hbm, w1_hbm, out_hbm,
                 a0, b0, a1, b1, row_t, o0, o1, r0_v, w0_v, w1_v,
                 in_sem0, in_sem1, out_sem0, out_sem1):
    cid = lax.axis_index("c")
    sid = lax.axis_index("s")
    wid = sid * 2 + cid
    batch = wid // _WPB
    i_base = (wid % _WPB) * _ROWS_PER_W
    in_base = batch * _H
    out_base = batch * _HD

    pltpu.sync_copy(r0_hbm, r0_v)
    pltpu.sync_copy(w0_hbm, w0_v)
    pltpu.sync_copy(w1_hbm, w1_v)

    def fetch(i, a, b, sem):
        r0 = r0_v[pl.ds(i, 16)][0]
        pltpu.async_copy(x_hbm.at[in_base + r0], a, sem)
        pltpu.async_copy(x_hbm.at[in_base + r0 + 1], b, sem)

    fetch(i_base, a0, b0, in_sem0)
    fetch(i_base + 1, a1, b1, in_sem1)

    def do_row(i, m, a, b, o, in_sem, out_sem):
        pltpu.make_async_copy(x_hbm.at[0], a, in_sem).wait()
        pltpu.make_async_copy(x_hbm.at[0], b, in_sem).wait()
        w0 = w0_v[pl.ds(i, 16)][0]
        w1 = w1_v[pl.ds(i, 16)][0]

        @plsc.parallel_loop(0, _W, unroll=4)
        def blend(k):
            off = k * _C
            for g in range(_CG):
                sl = pl.ds(off + g * 16, 16)
                row_t[sl] = w0 * a[sl] + w1 * b[sl]

        @pl.when(m >= 1)
        def _():
            pltpu.make_async_copy(o, out_hbm.at[0], out_sem).wait()

        @plsc.parallel_loop(0, _W - 3, unroll=4)
        def colpair(mm):
            k = mm + 2
            t0 = (k - 1) * _C
            t1 = k * _C
            o_odd = (2 * k - 1) * _C
            o_even = 2 * k * _C
            for g in range(_CG):
                gg = g * 16
                va = row_t[pl.ds(t0 + gg, 16)]
                vb = row_t[pl.ds(t1 + gg, 16)]
                o[pl.ds(o_odd + gg, 16)] = 0.75 * va + 0.25 * vb
                o[pl.ds(o_even + gg, 16)] = 0.25 * va + 0.75 * vb

        for j, cb, v0, v1 in _EDGE_COLS:
            for g in range(_CG):
                gg = g * 16
                o[pl.ds(j * _C + gg, 16)] = (
                    v0 * row_t[pl.ds(cb * _C + gg, 16)]
                    + v1 * row_t[pl.ds((cb + 1) * _C + gg, 16)])

        pltpu.async_copy(o, out_hbm.at[out_base + i], out_sem)

        @pl.when(m < _PAIRS - 1)
        def _():
            fetch(i + 2, a, b, in_sem)

    def per_pair(m, carry):
        i0 = i_base + 2 * m
        do_row(i0, m, a0, b0, o0, in_sem0, out_sem0)
        do_row(i0 + 1, m, a1, b1, o1, in_sem1, out_sem1)
        return carry

    lax.fori_loop(0, _PAIRS, per_pair, 0)
    pltpu.make_async_copy(o0, out_hbm.at[0], out_sem0).wait()
    pltpu.make_async_copy(o1, out_hbm.at[1], out_sem1).wait()


def kernel(inputs):
    x = inputs.reshape(_B * _H, _ROW_IN)
    out = _upsample_sc(
        x,
        jnp.asarray(_pad_tbl(_R0_NP)),
        jnp.asarray(_pad_tbl(_W0_NP)),
        jnp.asarray(_pad_tbl(_W1_NP)),
    )
    return out.reshape(_B, _HD, _WD, _C)


# speedup vs baseline: 12.8284x; 1.4456x over previous
import functools

import jax
import jax.numpy as jnp
import numpy as np
from jax import lax
from jax.experimental import pallas as pl
from jax.experimental.pallas import tpu as pltpu
from jax.experimental.pallas import tpu_sc as plsc

_STRIDES = 2
_H = 112
_W = 112
_C = 96
_B = 4
_HD = _H * _STRIDES
_WD = _W * _STRIDES
_NW = 32
_ROWS_PER_W = (_B * _HD) // _NW
_WPB = _HD // _ROWS_PER_W
_CG = _C // 16
_TBL = 240


def _interp_tables(src_size):
    s = float(src_size - 1)
    d = np.arange(2 * src_size, dtype=np.float64)
    low = (d - 1.0) / 1.5
    high = (d - 1.0 + 0.5 - (s - 1.0) * 2.0) / 1.5 + (s - 1.0)
    mid = (d - 1.0 + 0.5) / 2.0
    src = np.where(d < 2.5, low, np.where(d > 1.0 + (s - 1.0) * 2.0 - 0.5, high, mid))
    r0 = np.floor(src).astype(np.int64)
    fr = src - r0
    w0 = (1.0 - fr) * ((r0 >= 0) & (r0 < src_size))
    w1 = fr * ((r0 + 1 >= 0) & (r0 + 1 < src_size))
    base = np.clip(r0, 0, src_size - 2)
    tap0 = np.select([r0 < 0, r0 > src_size - 2], [w1, 0.0], w0)
    tap1 = np.select([r0 < 0, r0 > src_size - 2], [0.0, w0], w1)
    return (base.astype(np.int32), tap0.astype(np.float32),
            tap1.astype(np.float32))


_R0_NP, _W0_NP, _W1_NP = _interp_tables(_H)
_C0_NP, _V0_NP, _V1_NP = _interp_tables(_W)


def _pad_tbl(a):
    return np.pad(a, (0, _TBL - a.shape[0]))


_EDGE_COLS = [(j, int(_C0_NP[j]), float(_V0_NP[j]), float(_V1_NP[j]))
              for j in (0, 1, 2, _WD - 3, _WD - 2, _WD - 1)]

_PAIRS = _ROWS_PER_W // 2


@functools.partial(
    pl.kernel,
    mesh=plsc.VectorSubcoreMesh(core_axis_name="c", subcore_axis_name="s"),
    out_type=jax.ShapeDtypeStruct((_B, _HD, _WD, _C), jnp.float32),
    scratch_types=[
        pltpu.VMEM((_W, _C), jnp.float32),
        pltpu.VMEM((_W, _C), jnp.float32),
        pltpu.VMEM((_W, _C), jnp.float32),
        pltpu.VMEM((_W, _C), jnp.float32),
        pltpu.VMEM((_W, _C), jnp.float32),
        pltpu.VMEM((_WD, _C), jnp.float32),
        pltpu.VMEM((_WD, _C), jnp.float32),
        pltpu.VMEM((_TBL,), jnp.int32),
        pltpu.VMEM((_TBL,), jnp.float32),
        pltpu.VMEM((_TBL,), jnp.float32),
        pltpu.SemaphoreType.DMA,
        pltpu.SemaphoreType.DMA,
        pltpu.SemaphoreType.DMA,
        pltpu.SemaphoreType.DMA,
    ],
)
def _upsample_sc(x_hbm, r0_hbm, w0_hbm, w1_hbm, out_hbm,
                 a0, b0, a1, b1, row_t, o0, o1, r0_v, w0_v, w1_v,
                 in_sem0, in_sem1, out_sem0, out_sem1):
    cid = lax.axis_index("c")
    sid = lax.axis_index("s")
    wid = sid * 2 + cid
    batch = wid // _WPB
    i_base = (wid % _WPB) * _ROWS_PER_W

    pltpu.sync_copy(r0_hbm, r0_v)
    pltpu.sync_copy(w0_hbm, w0_v)
    pltpu.sync_copy(w1_hbm, w1_v)

    def fetch(i, a, b, sem):
        r0 = r0_v[pl.ds(i, 16)][0]
        pltpu.async_copy(x_hbm.at[batch, r0], a, sem)
        pltpu.async_copy(x_hbm.at[batch, r0 + 1], b, sem)

    fetch(i_base, a0, b0, in_sem0)
    fetch(i_base + 1, a1, b1, in_sem1)

    def do_row(i, m, a, b, o, in_sem, out_sem):
        pltpu.make_async_copy(x_hbm.at[0, 0], a, in_sem).wait()
        pltpu.make_async_copy(x_hbm.at[0, 0], b, in_sem).wait()
        w0 = w0_v[pl.ds(i, 16)][0]
        w1 = w1_v[pl.ds(i, 16)][0]

        @plsc.parallel_loop(0, _W, unroll=4)
        def blend(k):
            for g in range(_CG):
                sl = pl.ds(g * 16, 16)
                row_t[k, sl] = w0 * a[k, sl] + w1 * b[k, sl]

        @pl.when(m >= 1)
        def _():
            pltpu.make_async_copy(o, out_hbm.at[0, 0], out_sem).wait()

        @plsc.parallel_loop(0, _W - 3, unroll=4)
        def colpair(mm):
            k = mm + 2
            for g in range(_CG):
                sl = pl.ds(g * 16, 16)
                va = row_t[k - 1, sl]
                vb = row_t[k, sl]
                o[2 * k - 1, sl] = 0.75 * va + 0.25 * vb
                o[2 * k, sl] = 0.25 * va + 0.75 * vb

        for j, cb, v0, v1 in _EDGE_COLS:
            for g in range(_CG):
                sl = pl.ds(g * 16, 16)
                o[j, sl] = v0 * row_t[cb, sl] + v1 * row_t[cb + 1, sl]

        pltpu.async_copy(o, out_hbm.at[batch, i], out_sem)

        @pl.when(m < _PAIRS - 1)
        def _():
            fetch(i + 2, a, b, in_sem)

    def per_pair(m, carry):
        i0 = i_base + 2 * m
        do_row(i0, m, a0, b0, o0, in_sem0, out_sem0)
        do_row(i0 + 1, m, a1, b1, o1, in_sem1, out_sem1)
        return carry

    lax.fori_loop(0, _PAIRS, per_pair, 0)
    pltpu.make_async_copy(o0, out_hbm.at[0, 0], out_sem0).wait()
    pltpu.make_async_copy(o1, out_hbm.at[0, 1], out_sem1).wait()


def kernel(inputs):
    return _upsample_sc(
        inputs,
        jnp.asarray(_pad_tbl(_R0_NP)),
        jnp.asarray(_pad_tbl(_W0_NP)),
        jnp.asarray(_pad_tbl(_W1_NP)),
    )


# speedup vs baseline: 19.4617x; 1.5171x over previous
import functools

import jax
import jax.numpy as jnp
import numpy as np
from jax import lax
from jax.experimental import pallas as pl
from jax.experimental.pallas import tpu as pltpu
from jax.experimental.pallas import tpu_sc as plsc

_STRIDES = 2
_H = 112
_W = 112
_C = 96
_B = 4
_HD = _H * _STRIDES
_WD = _W * _STRIDES
_NW = 32
_ROWS_PER_W = (_B * _HD) // _NW
_WPB = _HD // _ROWS_PER_W
_WG = _W // 16
_WDG = _WD // 16
_TBL = 240


def _interp_tables(src_size):
    s = float(src_size - 1)
    d = np.arange(2 * src_size, dtype=np.float64)
    low = (d - 1.0) / 1.5
    high = (d - 1.0 + 0.5 - (s - 1.0) * 2.0) / 1.5 + (s - 1.0)
    mid = (d - 1.0 + 0.5) / 2.0
    src = np.where(d < 2.5, low, np.where(d > 1.0 + (s - 1.0) * 2.0 - 0.5, high, mid))
    r0 = np.floor(src).astype(np.int64)
    fr = src - r0
    w0 = (1.0 - fr) * ((r0 >= 0) & (r0 < src_size))
    w1 = fr * ((r0 + 1 >= 0) & (r0 + 1 < src_size))
    base = np.clip(r0, 0, src_size - 2)
    tap0 = np.select([r0 < 0, r0 > src_size - 2], [w1, 0.0], w0)
    tap1 = np.select([r0 < 0, r0 > src_size - 2], [0.0, w0], w1)
    return (base.astype(np.int32), tap0.astype(np.float32),
            tap1.astype(np.float32))


_R0_NP, _W0_NP, _W1_NP = _interp_tables(_H)
_C0_NP, _V0_NP, _V1_NP = _interp_tables(_W)


def _pad_tbl(a):
    return np.pad(a, (0, _TBL - a.shape[0]))


_PAIRS = _ROWS_PER_W // 2


@functools.partial(
    pl.kernel,
    mesh=plsc.VectorSubcoreMesh(core_axis_name="c", subcore_axis_name="s"),
    out_type=jax.ShapeDtypeStruct((_B, _HD, _C, _WD), jnp.float32),
    compiler_params=pltpu.CompilerParams(needs_layout_passes=False),
    scratch_types=[
        pltpu.VMEM((_C, _W), jnp.float32),
        pltpu.VMEM((_C, _W), jnp.float32),
        pltpu.VMEM((_C, _W), jnp.float32),
        pltpu.VMEM((_C, _W), jnp.float32),
        pltpu.VMEM((_C, _W), jnp.float32),
        pltpu.VMEM((_C, _WD), jnp.float32),
        pltpu.VMEM((_C, _WD), jnp.float32),
        pltpu.VMEM((_TBL,), jnp.int32),
        pltpu.VMEM((_TBL,), jnp.float32),
        pltpu.VMEM((_TBL,), jnp.float32),
        pltpu.VMEM((_WD,), jnp.int32),
        pltpu.VMEM((_WD,), jnp.float32),
        pltpu.VMEM((_WD,), jnp.float32),
        pltpu.SemaphoreType.DMA,
        pltpu.SemaphoreType.DMA,
        pltpu.SemaphoreType.DMA,
        pltpu.SemaphoreType.DMA,
    ],
)
def _upsample_sc(x_hbm, r0_hbm, w0_hbm, w1_hbm, c0_hbm, v0_hbm, v1_hbm,
                 out_hbm, a0, b0, a1, b1, row_t, o0, o1,
                 r0_v, w0_v, w1_v, c0_v, v0_v, v1_v,
                 in_sem0, in_sem1, out_sem0, out_sem1):
    cid = lax.axis_index("c")
    sid = lax.axis_index("s")
    wid = sid * 2 + cid
    batch = wid // _WPB
    i_base = (wid % _WPB) * _ROWS_PER_W

    pltpu.sync_copy(r0_hbm, r0_v)
    pltpu.sync_copy(w0_hbm, w0_v)
    pltpu.sync_copy(w1_hbm, w1_v)
    pltpu.sync_copy(c0_hbm, c0_v)
    pltpu.sync_copy(v0_hbm, v0_v)
    pltpu.sync_copy(v1_hbm, v1_v)

    def fetch(i, a, b, sem):
        r0 = r0_v[pl.ds(i, 16)][0]
        pltpu.async_copy(x_hbm.at[batch, r0], a, sem)
        pltpu.async_copy(x_hbm.at[batch, r0 + 1], b, sem)

    fetch(i_base, a0, b0, in_sem0)
    fetch(i_base + 1, a1, b1, in_sem1)

    def do_row(i, m, a, b, o, in_sem, out_sem):
        pltpu.make_async_copy(x_hbm.at[0, 0], a, in_sem).wait()
        pltpu.make_async_copy(x_hbm.at[0, 0], b, in_sem).wait()
        w0 = w0_v[pl.ds(i, 16)][0]
        w1 = w1_v[pl.ds(i, 16)][0]

        @plsc.parallel_loop(0, _C, unroll=2)
        def blend(c):
            for g in range(_WG):
                sl = pl.ds(g * 16, 16)
                row_t[c, sl] = w0 * a[c, sl] + w1 * b[c, sl]

        @pl.when(m >= 1)
        def _():
            pltpu.make_async_copy(o, out_hbm.at[0, 0], out_sem).wait()

        for blk in range(_WDG):
            sl = pl.ds(blk * 16, 16)
            idx0 = c0_v[sl]
            idx1 = idx0 + 1
            v0 = v0_v[sl]
            v1 = v1_v[sl]

            @plsc.parallel_loop(0, _C, unroll=2)
            def colc(c):
                cvec = jnp.full((16,), c, jnp.int32)
                t0 = plsc.load_gather(row_t, [cvec, idx0])
                t1 = plsc.load_gather(row_t, [cvec, idx1])
                o[c, sl] = v0 * t0 + v1 * t1

        pltpu.async_copy(o, out_hbm.at[batch, i], out_sem)

        @pl.when(m < _PAIRS - 1)
        def _():
            fetch(i + 2, a, b, in_sem)

    def per_pair(m, carry):
        i0 = i_base + 2 * m
        do_row(i0, m, a0, b0, o0, in_sem0, out_sem0)
        do_row(i0 + 1, m, a1, b1, o1, in_sem1, out_sem1)
        return carry

    lax.fori_loop(0, _PAIRS, per_pair, 0)
    pltpu.make_async_copy(o0, out_hbm.at[0, 0], out_sem0).wait()
    pltpu.make_async_copy(o1, out_hbm.at[0, 1], out_sem1).wait()


def kernel(inputs):
    x_t = jnp.transpose(inputs, (0, 1, 3, 2))
    out_t = _upsample_sc(
        x_t,
        jnp.asarray(_pad_tbl(_R0_NP)),
        jnp.asarray(_pad_tbl(_W0_NP)),
        jnp.asarray(_pad_tbl(_W1_NP)),
        jnp.asarray(_C0_NP),
        jnp.asarray(_V0_NP),
        jnp.asarray(_V1_NP),
    )
    return jnp.transpose(out_t, (0, 1, 3, 2))


# speedup vs baseline: 24.6691x; 1.2676x over previous
import functools

import jax
import jax.numpy as jnp
import numpy as np
from jax import lax
from jax.experimental import pallas as pl
from jax.experimental.pallas import tpu as pltpu
from jax.experimental.pallas import tpu_sc as plsc

_STRIDES = 2
_H = 112
_W = 112
_C = 96
_B = 4
_HD = _H * _STRIDES
_WD = _W * _STRIDES
_NW = 32
_ROWS_PER_W = (_B * _HD) // _NW
_WPB = _HD // _ROWS_PER_W
_WG = _W // 16
_WDG = _WD // 16
_TBL = 240


def _interp_tables(src_size):
    s = float(src_size - 1)
    d = np.arange(2 * src_size, dtype=np.float64)
    low = (d - 1.0) / 1.5
    high = (d - 1.0 + 0.5 - (s - 1.0) * 2.0) / 1.5 + (s - 1.0)
    mid = (d - 1.0 + 0.5) / 2.0
    src = np.where(d < 2.5, low, np.where(d > 1.0 + (s - 1.0) * 2.0 - 0.5, high, mid))
    r0 = np.floor(src).astype(np.int64)
    fr = src - r0
    w0 = (1.0 - fr) * ((r0 >= 0) & (r0 < src_size))
    w1 = fr * ((r0 + 1 >= 0) & (r0 + 1 < src_size))
    base = np.clip(r0, 0, src_size - 2)
    tap0 = np.select([r0 < 0, r0 > src_size - 2], [w1, 0.0], w0)
    tap1 = np.select([r0 < 0, r0 > src_size - 2], [0.0, w0], w1)
    return (base.astype(np.int32), tap0.astype(np.float32),
            tap1.astype(np.float32))


_R0_NP, _W0_NP, _W1_NP = _interp_tables(_H)
_C0_NP, _V0_NP, _V1_NP = _interp_tables(_W)


def _pad_tbl(a):
    return np.pad(a, (0, _TBL - a.shape[0]))


_PAIRS = _ROWS_PER_W // 2


@functools.partial(
    pl.kernel,
    mesh=plsc.VectorSubcoreMesh(core_axis_name="c", subcore_axis_name="s"),
    out_type=jax.ShapeDtypeStruct((_B, _HD, _C, _WD), jnp.float32),
    compiler_params=pltpu.CompilerParams(
        needs_layout_passes=False, skip_device_barrier=True),
    scratch_types=[
        pltpu.VMEM((_C, _W), jnp.float32),
        pltpu.VMEM((_C, _W), jnp.float32),
        pltpu.VMEM((_C, _W), jnp.float32),
        pltpu.VMEM((_C, _W), jnp.float32),
        pltpu.VMEM((_C * _W,), jnp.float32),
        pltpu.VMEM((_C, _WD), jnp.float32),
        pltpu.VMEM((_C, _WD), jnp.float32),
        pltpu.VMEM((_TBL,), jnp.int32),
        pltpu.VMEM((_TBL,), jnp.float32),
        pltpu.VMEM((_TBL,), jnp.float32),
        pltpu.VMEM((_WD,), jnp.int32),
        pltpu.VMEM((_WD,), jnp.float32),
        pltpu.VMEM((_WD,), jnp.float32),
        pltpu.SemaphoreType.DMA,
        pltpu.SemaphoreType.DMA,
        pltpu.SemaphoreType.DMA,
        pltpu.SemaphoreType.DMA,
    ],
)
def _upsample_sc(x_hbm, r0_hbm, w0_hbm, w1_hbm, c0_hbm, v0_hbm, v1_hbm,
                 out_hbm, a0, b0, a1, b1, row_t, o0, o1,
                 r0_v, w0_v, w1_v, c0_v, v0_v, v1_v,
                 in_sem0, in_sem1, out_sem0, out_sem1):
    cid = lax.axis_index("c")
    sid = lax.axis_index("s")
    wid = sid * 2 + cid
    batch = wid // _WPB
    i_base = (wid % _WPB) * _ROWS_PER_W

    pltpu.sync_copy(r0_hbm, r0_v)
    pltpu.sync_copy(w0_hbm, w0_v)
    pltpu.sync_copy(w1_hbm, w1_v)
    pltpu.sync_copy(c0_hbm, c0_v)
    pltpu.sync_copy(v0_hbm, v0_v)
    pltpu.sync_copy(v1_hbm, v1_v)

    def fetch(i, a, b, sem):
        r0 = r0_v[pl.ds(i, 16)][0]
        pltpu.async_copy(x_hbm.at[batch, r0], a, sem)
        pltpu.async_copy(x_hbm.at[batch, r0 + 1], b, sem)

    fetch(i_base, a0, b0, in_sem0)
    fetch(i_base + 1, a1, b1, in_sem1)

    def do_row(i, m, a, b, o, in_sem, out_sem):
        pltpu.make_async_copy(x_hbm.at[0, 0], a, in_sem).wait()
        pltpu.make_async_copy(x_hbm.at[0, 0], b, in_sem).wait()
        w0 = w0_v[pl.ds(i, 16)][0]
        w1 = w1_v[pl.ds(i, 16)][0]

        @plsc.parallel_loop(0, _C, unroll=4)
        def blend(c):
            base = c * _W
            for g in range(_WG):
                sl = pl.ds(g * 16, 16)
                row_t[pl.ds(base + g * 16, 16)] = w0 * a[c, sl] + w1 * b[c, sl]

        @pl.when(m >= 1)
        def _():
            pltpu.make_async_copy(o, out_hbm.at[0, 0], out_sem).wait()

        for blk in range(_WDG):
            sl = pl.ds(blk * 16, 16)
            idx0 = c0_v[sl]
            v0 = v0_v[sl]
            v1 = v1_v[sl]

            def colc(c, idxs):
                i0, i1 = idxs
                t0 = plsc.load_gather(row_t, [i0])
                t1 = plsc.load_gather(row_t, [i1])
                o[c, sl] = v0 * t0 + v1 * t1
                return (i0 + _W, i1 + _W)

            plsc.parallel_loop(0, _C, unroll=4,
                               carry=(idx0, idx0 + 1))(colc)

        pltpu.async_copy(o, out_hbm.at[batch, i], out_sem)

        @pl.when(m < _PAIRS - 1)
        def _():
            fetch(i + 2, a, b, in_sem)

    def per_pair(m, carry):
        i0 = i_base + 2 * m
        do_row(i0, m, a0, b0, o0, in_sem0, out_sem0)
        do_row(i0 + 1, m, a1, b1, o1, in_sem1, out_sem1)
        return carry

    lax.fori_loop(0, _PAIRS, per_pair, 0)
    pltpu.make_async_copy(o0, out_hbm.at[0, 0], out_sem0).wait()
    pltpu.make_async_copy(o1, out_hbm.at[0, 1], out_sem1).wait()


def kernel(inputs):
    x_t = jnp.transpose(inputs, (0, 1, 3, 2))
    out_t = _upsample_sc(
        x_t,
        jnp.asarray(_pad_tbl(_R0_NP)),
        jnp.asarray(_pad_tbl(_W0_NP)),
        jnp.asarray(_pad_tbl(_W1_NP)),
        jnp.asarray(_C0_NP),
        jnp.asarray(_V0_NP),
        jnp.asarray(_V1_NP),
    )
    return jnp.transpose(out_t, (0, 1, 3, 2))


# speedup vs baseline: 25.5834x; 1.0371x over previous
import functools

import jax
import jax.numpy as jnp
import numpy as np
from jax import lax
from jax.experimental import pallas as pl
from jax.experimental.pallas import tpu as pltpu
from jax.experimental.pallas import tpu_sc as plsc

_STRIDES = 2
_H = 112
_W = 112
_C = 96
_B = 4
_HD = _H * _STRIDES
_WD = _W * _STRIDES
_NW = 32
_ROWS_PER_W = (_B * _HD) // _NW
_WPB = _HD // _ROWS_PER_W
_WG = _W // 16
_WDG = _WD // 16


def _interp_tables(src_size):
    s = float(src_size - 1)
    d = np.arange(2 * src_size, dtype=np.float64)
    low = (d - 1.0) / 1.5
    high = (d - 1.0 + 0.5 - (s - 1.0) * 2.0) / 1.5 + (s - 1.0)
    mid = (d - 1.0 + 0.5) / 2.0
    src = np.where(d < 2.5, low, np.where(d > 1.0 + (s - 1.0) * 2.0 - 0.5, high, mid))
    r0 = np.floor(src).astype(np.int64)
    fr = src - r0
    w0 = (1.0 - fr) * ((r0 >= 0) & (r0 < src_size))
    w1 = fr * ((r0 + 1 >= 0) & (r0 + 1 < src_size))
    base = np.clip(r0, 0, src_size - 2)
    tap0 = np.select([r0 < 0, r0 > src_size - 2], [w1, 0.0], w0)
    tap1 = np.select([r0 < 0, r0 > src_size - 2], [0.0, w0], w1)
    return (base.astype(np.int32), tap0.astype(np.float32),
            tap1.astype(np.float32))


_R0_NP, _W0_NP, _W1_NP = _interp_tables(_H)
_C0_NP, _V0_NP, _V1_NP = _interp_tables(_W)

_ROW_SPECIALS = [(i, int(_R0_NP[i]), float(_W0_NP[i]), float(_W1_NP[i]))
                 for i in (0, 1, 2, _HD - 3, _HD - 2, _HD - 1)]
_COL_SPECIALS = [(j, int(_C0_NP[j]), float(_V0_NP[j]), float(_V1_NP[j]))
                 for j in (0, 1, 2, _WD - 3, _WD - 2, _WD - 1)]

_PAIRS = _ROWS_PER_W // 2


@functools.partial(
    pl.kernel,
    mesh=plsc.VectorSubcoreMesh(core_axis_name="c", subcore_axis_name="s"),
    out_type=jax.ShapeDtypeStruct((_B, _HD, _C, _WD), jnp.float32),
    compiler_params=pltpu.CompilerParams(
        needs_layout_passes=False, skip_device_barrier=True),
    scratch_types=[
        pltpu.VMEM((_C, _W), jnp.float32),
        pltpu.VMEM((_C, _W), jnp.float32),
        pltpu.VMEM((_C, _W), jnp.float32),
        pltpu.VMEM((_C, _W), jnp.float32),
        pltpu.VMEM((_C * _W,), jnp.float32),
        pltpu.VMEM((_C, _WD), jnp.float32),
        pltpu.VMEM((_C, _WD), jnp.float32),
        pltpu.SemaphoreType.DMA,
        pltpu.SemaphoreType.DMA,
        pltpu.SemaphoreType.DMA,
        pltpu.SemaphoreType.DMA,
    ],
)
def _upsample_sc(x_hbm, out_hbm, a0, b0, a1, b1, row_t, o0, o1,
                 in_sem0, in_sem1, out_sem0, out_sem1):
    cid = lax.axis_index("c")
    sid = lax.axis_index("s")
    wid = sid * 2 + cid
    batch = wid // _WPB
    i_base = (wid % _WPB) * _ROWS_PER_W

    def row_params(i):
        odd = i & 1
        k = i >> 1
        r0 = k - 1 + odd
        w0 = jnp.where(odd == 1, jnp.float32(0.75), jnp.float32(0.25))
        w1 = jnp.where(odd == 1, jnp.float32(0.25), jnp.float32(0.75))
        for si, sr0, sw0, sw1 in _ROW_SPECIALS:
            hit = i == si
            r0 = jnp.where(hit, sr0, r0)
            w0 = jnp.where(hit, jnp.float32(sw0), w0)
            w1 = jnp.where(hit, jnp.float32(sw1), w1)
        return r0, w0, w1

    def fetch(i, a, b, sem):
        r0, _, _ = row_params(i)
        pltpu.async_copy(x_hbm.at[batch, r0], a, sem)
        pltpu.async_copy(x_hbm.at[batch, r0 + 1], b, sem)

    fetch(i_base, a0, b0, in_sem0)
    fetch(i_base + 1, a1, b1, in_sem1)

    def do_row(i, m, a, b, o, in_sem, out_sem):
        pltpu.make_async_copy(x_hbm.at[0, 0], a, in_sem).wait()
        pltpu.make_async_copy(x_hbm.at[0, 0], b, in_sem).wait()
        _, w0, w1 = row_params(i)

        @plsc.parallel_loop(0, _C, unroll=4)
        def blend(c):
            base = c * _W
            for g in range(_WG):
                sl = pl.ds(g * 16, 16)
                row_t[pl.ds(base + g * 16, 16)] = w0 * a[c, sl] + w1 * b[c, sl]

        @pl.when(m >= 1)
        def _():
            pltpu.make_async_copy(o, out_hbm.at[0, 0], out_sem).wait()

        jl = lax.iota(jnp.int32, 16)
        for blk in range(_WDG):
            sl = pl.ds(blk * 16, 16)
            j = jl + blk * 16
            odd = (j & 1) == 1
            idx0 = jnp.maximum(jnp.minimum((j - 1) >> 1, _W - 2), 0)
            v0 = jnp.where(odd, jnp.float32(0.75), jnp.float32(0.25))
            v1 = jnp.where(odd, jnp.float32(0.25), jnp.float32(0.75))
            if blk in (0, _WDG - 1):
                for sj, sc0, sv0, sv1 in _COL_SPECIALS:
                    hit = j == sj
                    idx0 = jnp.where(hit, sc0, idx0)
                    v0 = jnp.where(hit, jnp.float32(sv0), v0)
                    v1 = jnp.where(hit, jnp.float32(sv1), v1)

            def colc(c, idxs):
                i0, i1 = idxs
                t0 = plsc.load_gather(row_t, [i0])
                t1 = plsc.load_gather(row_t, [i1])
                o[c, sl] = v0 * t0 + v1 * t1
                return (i0 + _W, i1 + _W)

            plsc.parallel_loop(0, _C, unroll=4,
                               carry=(idx0, idx0 + 1))(colc)

        pltpu.async_copy(o, out_hbm.at[batch, i], out_sem)

        @pl.when(m < _PAIRS - 1)
        def _():
            fetch(i + 2, a, b, in_sem)

    def per_pair(m, carry):
        i0 = i_base + 2 * m
        do_row(i0, m, a0, b0, o0, in_sem0, out_sem0)
        do_row(i0 + 1, m, a1, b1, o1, in_sem1, out_sem1)
        return carry

    lax.fori_loop(0, _PAIRS, per_pair, 0)
    pltpu.make_async_copy(o0, out_hbm.at[0, 0], out_sem0).wait()
    pltpu.make_async_copy(o1, out_hbm.at[0, 1], out_sem1).wait()


def kernel(inputs):
    x_t = jnp.transpose(inputs, (0, 1, 3, 2))
    out_t = _upsample_sc(x_t)
    return jnp.transpose(out_t, (0, 1, 3, 2))


# speedup vs baseline: 29.9706x; 1.1715x over previous
import functools

import jax
import jax.numpy as jnp
import numpy as np
from jax import lax
from jax.experimental import pallas as pl
from jax.experimental.pallas import tpu as pltpu
from jax.experimental.pallas import tpu_sc as plsc

_STRIDES = 2
_H = 112
_W = 112
_C = 96
_B = 4
_HD = _H * _STRIDES
_WD = _W * _STRIDES
_NW = 32
_ROWS_PER_W = (_B * _HD) // _NW
_WPB = _HD // _ROWS_PER_W
_WG = _W // 16
_WDG = _WD // 16


def _interp_tables(src_size):
    s = float(src_size - 1)
    d = np.arange(2 * src_size, dtype=np.float64)
    low = (d - 1.0) / 1.5
    high = (d - 1.0 + 0.5 - (s - 1.0) * 2.0) / 1.5 + (s - 1.0)
    mid = (d - 1.0 + 0.5) / 2.0
    src = np.where(d < 2.5, low, np.where(d > 1.0 + (s - 1.0) * 2.0 - 0.5, high, mid))
    r0 = np.floor(src).astype(np.int64)
    fr = src - r0
    w0 = (1.0 - fr) * ((r0 >= 0) & (r0 < src_size))
    w1 = fr * ((r0 + 1 >= 0) & (r0 + 1 < src_size))
    base = np.clip(r0, 0, src_size - 2)
    tap0 = np.select([r0 < 0, r0 > src_size - 2], [w1, 0.0], w0)
    tap1 = np.select([r0 < 0, r0 > src_size - 2], [0.0, w0], w1)
    return (base.astype(np.int32), tap0.astype(np.float32),
            tap1.astype(np.float32))


_R0_NP, _W0_NP, _W1_NP = _interp_tables(_H)
_C0_NP, _V0_NP, _V1_NP = _interp_tables(_W)

_ROW_SPECIALS = [(i, int(_R0_NP[i]), float(_W0_NP[i]), float(_W1_NP[i]))
                 for i in (0, 1, 2, _HD - 3, _HD - 2, _HD - 1)]
_COL_SPECIALS = [(j, int(_C0_NP[j]), float(_V0_NP[j]), float(_V1_NP[j]))
                 for j in (0, 1, 2, _WD - 3, _WD - 2, _WD - 1)]

_PAIRS = _ROWS_PER_W // 2


@functools.partial(
    pl.kernel,
    mesh=plsc.VectorSubcoreMesh(core_axis_name="c", subcore_axis_name="s"),
    out_type=jax.ShapeDtypeStruct((_B, _HD, _C, _WD), jnp.float32),
    compiler_params=pltpu.CompilerParams(
        needs_layout_passes=False, skip_device_barrier=True),
    scratch_types=[
        pltpu.VMEM((_C, _W), jnp.float32),
        pltpu.VMEM((_C, _W), jnp.float32),
        pltpu.VMEM((_C, _W), jnp.float32),
        pltpu.VMEM((_C, _W), jnp.float32),
        pltpu.VMEM((_C * _W,), jnp.float32),
        pltpu.VMEM((_C, _WD), jnp.float32),
        pltpu.VMEM((_C, _WD), jnp.float32),
        pltpu.SemaphoreType.DMA,
        pltpu.SemaphoreType.DMA,
        pltpu.SemaphoreType.DMA,
        pltpu.SemaphoreType.DMA,
    ],
)
def _upsample_sc(x_hbm, out_hbm, a0, b0, a1, b1, row_t, o0, o1,
                 in_sem0, in_sem1, out_sem0, out_sem1):
    cid = lax.axis_index("c")
    sid = lax.axis_index("s")
    wid = sid * 2 + cid
    batch = wid // _WPB
    i_base = (wid % _WPB) * _ROWS_PER_W

    def row_params(i):
        odd = i & 1
        k = i >> 1
        r0 = k - 1 + odd
        w0 = jnp.where(odd == 1, jnp.float32(0.75), jnp.float32(0.25))
        w1 = jnp.where(odd == 1, jnp.float32(0.25), jnp.float32(0.75))
        for si, sr0, sw0, sw1 in _ROW_SPECIALS:
            hit = i == si
            r0 = jnp.where(hit, sr0, r0)
            w0 = jnp.where(hit, jnp.float32(sw0), w0)
            w1 = jnp.where(hit, jnp.float32(sw1), w1)
        return r0, w0, w1

    def fetch(i, a, b, sem):
        r0, _, _ = row_params(i)
        pltpu.async_copy(x_hbm.at[batch, r0], a, sem)
        pltpu.async_copy(x_hbm.at[batch, r0 + 1], b, sem)

    fetch(i_base, a0, b0, in_sem0)
    fetch(i_base + 1, a1, b1, in_sem1)

    jl = lax.iota(jnp.int32, 16)
    zero_v = jl & 0
    fifteen_v = zero_v + 15
    lane0 = jl == 0
    lane15 = jl == 15
    rel_e0 = jnp.maximum((jl - 1) >> 1, 0)
    rel_e1 = (jl + 1) >> 1
    rel_o0 = ((jl + 1) >> 1) + 7
    rel_o1c = jnp.minimum(rel_o0 + 1, 15)
    odd_l = (jl & 1) == 1
    v0_int = jnp.where(odd_l, jnp.float32(0.75), jnp.float32(0.25))
    v1_int = jnp.where(odd_l, jnp.float32(0.25), jnp.float32(0.75))
    v0_b0, v1_b0 = v0_int, v1_int
    v0_b13, v1_b13 = v0_int, v1_int
    idx0_b13 = rel_o0
    for sj, sc0, sv0, sv1 in _COL_SPECIALS:
        if sj < 16:
            hit = jl == sj
            v0_b0 = jnp.where(hit, jnp.float32(sv0), v0_b0)
            v1_b0 = jnp.where(hit, jnp.float32(sv1), v1_b0)
        else:
            hit = jl == (sj - (_WD - 16))
            v0_b13 = jnp.where(hit, jnp.float32(sv0), v0_b13)
            v1_b13 = jnp.where(hit, jnp.float32(sv1), v1_b13)
            idx0_b13 = jnp.where(hit, sc0 - 16 * (_WG - 1), idx0_b13)

    def _dg(vec, idx):
        return jnp.take_along_axis(vec, idx, axis=0)

    _MEM_BLOCKS = (1, 3, 5, 7, 9)

    def do_row(i, m, a, b, o, in_sem, out_sem):
        pltpu.make_async_copy(x_hbm.at[0, 0], a, in_sem).wait()
        pltpu.make_async_copy(x_hbm.at[0, 0], b, in_sem).wait()
        _, w0, w1 = row_params(i)

        @pl.when(m >= 1)
        def _():
            pltpu.make_async_copy(o, out_hbm.at[0, 0], out_sem).wait()

        @plsc.parallel_loop(0, _C, unroll=2)
        def fused(c):
            base = c * _W
            t = []
            for g in range(_WG):
                sl = pl.ds(g * 16, 16)
                tg = w0 * a[c, sl] + w1 * b[c, sl]
                t.append(tg)
                row_t[pl.ds(base + g * 16, 16)] = tg
            s0 = rel_o0 + base
            for blk in range(_WDG):
                g = blk // 2
                if blk == 0:
                    t0 = _dg(t[0], rel_e0)
                    t1 = _dg(t[0], rel_e1)
                    v0, v1 = v0_b0, v1_b0
                elif blk == _WDG - 1:
                    t0 = _dg(t[_WG - 1], idx0_b13)
                    t1 = _dg(t[_WG - 1], rel_o1c)
                    v0, v1 = v0_b13, v1_b13
                elif blk in _MEM_BLOCKS:
                    t0 = plsc.load_gather(row_t, [s0 + 16 * g])
                    t1 = plsc.load_gather(row_t, [s0 + (16 * g + 1)])
                    v0, v1 = v0_int, v1_int
                elif blk % 2 == 0:
                    t0 = jnp.where(lane0, _dg(t[g - 1], fifteen_v),
                                   _dg(t[g], rel_e0))
                    t1 = _dg(t[g], rel_e1)
                    v0, v1 = v0_int, v1_int
                else:
                    t0 = _dg(t[g], rel_o0)
                    t1 = jnp.where(lane15, _dg(t[g + 1], zero_v),
                                   _dg(t[g], rel_o1c))
                    v0, v1 = v0_int, v1_int
                o[c, pl.ds(blk * 16, 16)] = v0 * t0 + v1 * t1

        pltpu.async_copy(o, out_hbm.at[batch, i], out_sem)

        @pl.when(m < _PAIRS - 1)
        def _():
            fetch(i + 2, a, b, in_sem)

    def per_pair(m, carry):
        i0 = i_base + 2 * m
        do_row(i0, m, a0, b0, o0, in_sem0, out_sem0)
        do_row(i0 + 1, m, a1, b1, o1, in_sem1, out_sem1)
        return carry

    lax.fori_loop(0, _PAIRS, per_pair, 0)
    pltpu.make_async_copy(o0, out_hbm.at[0, 0], out_sem0).wait()
    pltpu.make_async_copy(o1, out_hbm.at[0, 1], out_sem1).wait()


def kernel(inputs):
    x_t = jnp.transpose(inputs, (0, 1, 3, 2))
    out_t = _upsample_sc(x_t)
    return jnp.transpose(out_t, (0, 1, 3, 2))
